# baseline (speedup 1.0000x reference)
import functools
import jax
import jax.numpy as jnp
from jax import lax
from jax.experimental import pallas as pl
from jax.experimental.pallas import tpu as pltpu

_NUM_HEADS = 16


def _round_up(x, m):
    return (x + m - 1) // m * m


def _qkvt_kernel(x_ref, wt_ref, qkvt_ref):
    x = x_ref[0].astype(jnp.bfloat16)
    acc = lax.dot_general(wt_ref[...], x, (((1,), (1,)), ((), ())),
                          preferred_element_type=jnp.float32)
    qkvt_ref[0] = acc.astype(qkvt_ref.dtype)


def _attn_kernel(q_ref, k_ref, v_ref, wo_ref, bo_ref, o_ref,
                 st_ref, mt_ref, l_ref, ctx_ref, *, num_heads, head_dim, tk):
    qi = pl.program_id(1)
    tq = q_ref.shape[2]
    T = k_ref.shape[2]
    n_k = T // tk
    n_q = T // tq
    r = tq // tk

    qt = q_ref[0]

    i0 = lax.broadcasted_iota(jnp.int32, (tk, tq), 0)
    i1 = lax.broadcasted_iota(jnp.int32, (tk, tq), 1)
    negs = [jnp.where(o * tk + i0 > i1, -jnp.inf, 0.0).astype(jnp.float32)
            for o in range(r)]

    mt_ref[...] = jnp.full_like(mt_ref, -jnp.inf)

    def _tile_scores(j, o):
        for h in range(num_heads):
            lo = h * head_dim
            kt_h = k_ref[0, lo:lo + head_dim, j * tk:(j + 1) * tk]
            st = lax.dot_general(kt_h, qt[lo:lo + head_dim, :],
                                 (((0,), (0,)), ((), ())),
                                 preferred_element_type=jnp.float32)
            if o is not None:
                st = st + negs[o]
            st_ref[h, j * tk:(j + 1) * tk, :] = st
            mt_ref[j, h:h + 1, :] = jnp.max(st, axis=0, keepdims=True)

    for j in range(n_k):
        for o in range(r):
            if (j - o) % r == 0 and 0 <= (j - o) // r < n_q:
                @pl.when(j == r * qi + o)
                def _():
                    _tile_scores(j, o)
        if j < r * (n_q - 1):
            @pl.when(j < r * qi)
            def _():
                _tile_scores(j, None)

    m_all = mt_ref[0]
    for j in range(1, n_k):
        m_all = jnp.maximum(m_all, mt_ref[j])

    def _tile_accum(j, first):
        for h in range(num_heads):
            lo = h * head_dim
            p = jnp.exp(st_ref[h, j * tk:(j + 1) * tk, :]
                        - m_all[h:h + 1, :])
            psum = jnp.sum(p, axis=0, keepdims=True)
            vt_h = v_ref[0, lo:lo + head_dim, j * tk:(j + 1) * tk]
            pv = lax.dot_general(vt_h, p.astype(jnp.bfloat16),
                                 (((1,), (0,)), ((), ())),
                                 preferred_element_type=jnp.float32)
            if first:
                l_ref[h:h + 1, :] = psum
                ctx_ref[lo:lo + head_dim, :] = pv
            else:
                l_ref[h:h + 1, :] = l_ref[h:h + 1, :] + psum
                ctx_ref[lo:lo + head_dim, :] = ctx_ref[lo:lo + head_dim, :] + pv

    for j in range(n_k):
        if j < r:
            _tile_accum(j, j == 0)
        else:
            @pl.when(j <= r * qi + r - 1)
            def _():
                _tile_accum(j, False)

    pieces = []
    for h in range(num_heads):
        lo = h * head_dim
        inv_l = pl.reciprocal(l_ref[h:h + 1, :], approx=False)
        pieces.append(ctx_ref[lo:lo + head_dim, :] * inv_l)
    ctx = jnp.concatenate(pieces, axis=0).astype(jnp.bfloat16)
    out = lax.dot_general(ctx, wo_ref[...], (((0,), (0,)), ((), ())),
                          preferred_element_type=jnp.float32)
    o_ref[0] = (out + bo_ref[...].astype(jnp.float32)).astype(o_ref.dtype)


def kernel(x, wq, wk, wv, wo, bo):
    B, T, d_in = x.shape
    d_out = wq.shape[1]
    num_heads = _NUM_HEADS
    head_dim = d_out // num_heads
    scale = 1.0 / (head_dim ** 0.5)

    wqkvt = jnp.concatenate([wq * scale, wk, wv], axis=1).T.astype(jnp.bfloat16)
    bo2 = bo.reshape(1, d_out)

    tp = min(1024, _round_up(T, 8))
    t = min(512, _round_up(T, 8))
    tk = min(256, t)
    T_pad = _round_up(T, max(tp, t))
    if T_pad != T:
        x = jnp.pad(x, ((0, 0), (0, T_pad - T), (0, 0)))
    n_p = T_pad // tp
    n_t = T_pad // t

    qkvt = pl.pallas_call(
        _qkvt_kernel,
        out_shape=jax.ShapeDtypeStruct((B, 3 * d_out, T_pad), jnp.bfloat16),
        grid=(B, n_p),
        in_specs=[
            pl.BlockSpec((1, tp, d_in), lambda b, i: (b, i, 0)),
            pl.BlockSpec((3 * d_out, d_in), lambda b, i: (0, 0)),
        ],
        out_specs=pl.BlockSpec((1, 3 * d_out, tp), lambda b, i: (b, 0, i)),
        compiler_params=pltpu.CompilerParams(
            dimension_semantics=("parallel", "parallel")),
    )(x, wqkvt)

    out = pl.pallas_call(
        functools.partial(_attn_kernel, num_heads=num_heads,
                          head_dim=head_dim, tk=tk),
        out_shape=jax.ShapeDtypeStruct((B, T_pad, d_out), x.dtype),
        grid=(B, n_t),
        in_specs=[
            pl.BlockSpec((1, d_out, t), lambda b, qi: (b, 0, qi)),
            pl.BlockSpec((1, d_out, T_pad), lambda b, qi: (b, 1, 0)),
            pl.BlockSpec((1, d_out, T_pad), lambda b, qi: (b, 2, 0)),
            pl.BlockSpec((d_out, d_out), lambda b, qi: (0, 0)),
            pl.BlockSpec((1, d_out), lambda b, qi: (0, 0)),
        ],
        out_specs=pl.BlockSpec((1, t, d_out), lambda b, qi: (b, qi, 0)),
        scratch_shapes=[
            pltpu.VMEM((num_heads, T_pad, t), jnp.float32),
            pltpu.VMEM((T_pad // tk, num_heads, t), jnp.float32),
            pltpu.VMEM((num_heads, t), jnp.float32),
            pltpu.VMEM((d_out, t), jnp.float32),
        ],
        compiler_params=pltpu.CompilerParams(
            dimension_semantics=("parallel", "arbitrary")),
    )(qkvt, qkvt, qkvt, wo.astype(jnp.bfloat16), bo2)

    if T_pad != T:
        out = out[:, :T, :]
    return out


# speedup vs baseline: 1.0002x; 1.0002x over previous
import functools
import jax
import jax.numpy as jnp
from jax import lax
from jax.experimental import pallas as pl
from jax.experimental.pallas import tpu as pltpu

_NUM_HEADS = 16


def _round_up(x, m):
    return (x + m - 1) // m * m


def _qkvt_kernel(x_ref, wt_ref, qkvt_ref):
    x = x_ref[0].astype(jnp.bfloat16)
    acc = lax.dot_general(wt_ref[...], x, (((1,), (1,)), ((), ())),
                          preferred_element_type=jnp.float32)
    qkvt_ref[0] = acc.astype(qkvt_ref.dtype)


def _attn_kernel(q_ref, k_ref, v_ref, wo_ref, bo_ref, o_ref,
                 st_ref, mt_ref, l_ref, ctx_ref, *, num_heads, head_dim, tk):
    qi = pl.program_id(1)
    tq = q_ref.shape[2]
    T = k_ref.shape[2]
    n_k = T // tk

    qt = q_ref[0]

    neg_diag = jnp.where(
        lax.broadcasted_iota(jnp.int32, (tk, tq), 0)
        > lax.broadcasted_iota(jnp.int32, (tk, tq), 1),
        -jnp.inf, 0.0).astype(jnp.float32)

    mt_ref[...] = jnp.full_like(mt_ref, -jnp.inf)

    def _tile_scores(j, masked):
        for h in range(num_heads):
            lo = h * head_dim
            kt_h = k_ref[0, lo:lo + head_dim, j * tk:(j + 1) * tk]
            st = lax.dot_general(kt_h, qt[lo:lo + head_dim, :],
                                 (((0,), (0,)), ((), ())),
                                 preferred_element_type=jnp.float32)
            if masked:
                st = st + neg_diag
            st_ref[h, j * tk:(j + 1) * tk, :] = st
            mt_ref[j, h:h + 1, :] = jnp.max(st, axis=0, keepdims=True)

    for j in range(n_k):
        if j == 0:
            @pl.when(qi == 0)
            def _():
                _tile_scores(0, True)

            @pl.when(qi > 0)
            def _():
                _tile_scores(0, False)
        else:
            @pl.when(j == qi)
            def _():
                _tile_scores(j, True)

            @pl.when(j < qi)
            def _():
                _tile_scores(j, False)

    m_all = mt_ref[0]
    for j in range(1, n_k):
        m_all = jnp.maximum(m_all, mt_ref[j])

    def _tile_accum(j, first):
        for h in range(num_heads):
            lo = h * head_dim
            p = jnp.exp(st_ref[h, j * tk:(j + 1) * tk, :]
                        - m_all[h:h + 1, :])
            psum = jnp.sum(p, axis=0, keepdims=True)
            vt_h = v_ref[0, lo:lo + head_dim, j * tk:(j + 1) * tk]
            pv = lax.dot_general(vt_h, p.astype(jnp.bfloat16),
                                 (((1,), (0,)), ((), ())),
                                 preferred_element_type=jnp.float32)
            if first:
                l_ref[h:h + 1, :] = psum
                ctx_ref[lo:lo + head_dim, :] = pv
            else:
                l_ref[h:h + 1, :] = l_ref[h:h + 1, :] + psum
                ctx_ref[lo:lo + head_dim, :] = ctx_ref[lo:lo + head_dim, :] + pv

    _tile_accum(0, True)
    for j in range(1, n_k):
        @pl.when(j <= qi)
        def _():
            _tile_accum(j, False)

    pieces = []
    for h in range(num_heads):
        lo = h * head_dim
        inv_l = pl.reciprocal(l_ref[h:h + 1, :], approx=False)
        pieces.append(ctx_ref[lo:lo + head_dim, :] * inv_l)
    ctx = jnp.concatenate(pieces, axis=0).astype(jnp.bfloat16)
    out = lax.dot_general(ctx, wo_ref[...], (((0,), (0,)), ((), ())),
                          preferred_element_type=jnp.float32)
    o_ref[0] = (out + bo_ref[...].astype(jnp.float32)).astype(o_ref.dtype)


def kernel(x, wq, wk, wv, wo, bo):
    B, T, d_in = x.shape
    d_out = wq.shape[1]
    num_heads = _NUM_HEADS
    head_dim = d_out // num_heads
    scale = 1.0 / (head_dim ** 0.5)

    wqkvt = jnp.concatenate([wq * scale, wk, wv], axis=1).T.astype(jnp.bfloat16)
    bo2 = bo.reshape(1, d_out)

    t = min(256, _round_up(T, 8))
    tk = t
    T_pad = _round_up(T, t)
    tp = next(c for c in (1024, 512, 256, t) if T_pad % c == 0)
    if T_pad != T:
        x = jnp.pad(x, ((0, 0), (0, T_pad - T), (0, 0)))
    n_p = T_pad // tp
    n_t = T_pad // t

    qkvt = pl.pallas_call(
        _qkvt_kernel,
        out_shape=jax.ShapeDtypeStruct((B, 3 * d_out, T_pad), jnp.bfloat16),
        grid=(B, n_p),
        in_specs=[
            pl.BlockSpec((1, tp, d_in), lambda b, i: (b, i, 0)),
            pl.BlockSpec((3 * d_out, d_in), lambda b, i: (0, 0)),
        ],
        out_specs=pl.BlockSpec((1, 3 * d_out, tp), lambda b, i: (b, 0, i)),
        compiler_params=pltpu.CompilerParams(
            dimension_semantics=("parallel", "parallel")),
    )(x, wqkvt)

    out = pl.pallas_call(
        functools.partial(_attn_kernel, num_heads=num_heads,
                          head_dim=head_dim, tk=tk),
        out_shape=jax.ShapeDtypeStruct((B, T_pad, d_out), x.dtype),
        grid=(B, n_t),
        in_specs=[
            pl.BlockSpec((1, d_out, t), lambda b, qi: (b, 0, qi)),
            pl.BlockSpec((1, d_out, T_pad), lambda b, qi: (b, 1, 0)),
            pl.BlockSpec((1, d_out, T_pad), lambda b, qi: (b, 2, 0)),
            pl.BlockSpec((d_out, d_out), lambda b, qi: (0, 0)),
            pl.BlockSpec((1, d_out), lambda b, qi: (0, 0)),
        ],
        out_specs=pl.BlockSpec((1, t, d_out), lambda b, qi: (b, qi, 0)),
        scratch_shapes=[
            pltpu.VMEM((num_heads, T_pad, t), jnp.float32),
            pltpu.VMEM((T_pad // tk, num_heads, t), jnp.float32),
            pltpu.VMEM((num_heads, t), jnp.float32),
            pltpu.VMEM((d_out, t), jnp.float32),
        ],
        compiler_params=pltpu.CompilerParams(
            dimension_semantics=("parallel", "arbitrary")),
    )(qkvt, qkvt, qkvt, wo.astype(jnp.bfloat16), bo2)

    if T_pad != T:
        out = out[:, :T, :]
    return out


# speedup vs baseline: 1.0033x; 1.0031x over previous
import functools
import jax
import jax.numpy as jnp
from jax import lax
from jax.experimental import pallas as pl
from jax.experimental.pallas import tpu as pltpu

_NUM_HEADS = 16


def _round_up(x, m):
    return (x + m - 1) // m * m


def _qkvt_kernel(x_ref, wt_ref, qkvt_ref):
    x = x_ref[0].astype(jnp.bfloat16)
    acc = lax.dot_general(wt_ref[...], x, (((1,), (1,)), ((), ())),
                          preferred_element_type=jnp.float32)
    qkvt_ref[0] = acc.astype(qkvt_ref.dtype)


def _attn_kernel(q_ref, k_ref, v_ref, wo_ref, bo_ref, o_ref,
                 st_ref, mt_ref, l_ref, ctx_ref, *, num_heads, head_dim, tk):
    qi = pl.program_id(1)
    tq = q_ref.shape[2]
    T = k_ref.shape[2]
    n_k = T // tk

    qt = q_ref[0]

    neg_diag = jnp.where(
        lax.broadcasted_iota(jnp.int32, (tk, tq), 0)
        > lax.broadcasted_iota(jnp.int32, (tk, tq), 1),
        -jnp.inf, 0.0).astype(jnp.float32)

    mt_ref[...] = jnp.full_like(mt_ref, -jnp.inf)

    def _tile_scores(j, masked):
        for h in range(num_heads):
            lo = h * head_dim
            kt_h = k_ref[0, lo:lo + head_dim, j * tk:(j + 1) * tk]
            st = lax.dot_general(kt_h, qt[lo:lo + head_dim, :],
                                 (((0,), (0,)), ((), ())),
                                 preferred_element_type=jnp.float32)
            if masked:
                st = st + neg_diag
            st_ref[h, j * tk:(j + 1) * tk, :] = st
            mt_ref[j, h:h + 1, :] = jnp.max(st, axis=0, keepdims=True)

    for j in range(n_k):
        if j == 0:
            @pl.when(qi == 0)
            def _():
                _tile_scores(0, True)

            @pl.when(qi > 0)
            def _():
                _tile_scores(0, False)
        else:
            @pl.when(j == qi)
            def _():
                _tile_scores(j, True)

            @pl.when(j < qi)
            def _():
                _tile_scores(j, False)

    m_all = mt_ref[0]
    for j in range(1, n_k):
        m_all = jnp.maximum(m_all, mt_ref[j])

    def _tile_accum(j, first):
        for h in range(num_heads):
            lo = h * head_dim
            p = jnp.exp(st_ref[h, j * tk:(j + 1) * tk, :]
                        - m_all[h:h + 1, :])
            psum = jnp.sum(p, axis=0, keepdims=True)
            vt_h = v_ref[0, lo:lo + head_dim, j * tk:(j + 1) * tk]
            pv = lax.dot_general(vt_h, p.astype(jnp.bfloat16),
                                 (((1,), (0,)), ((), ())),
                                 preferred_element_type=jnp.float32)
            if first:
                l_ref[h:h + 1, :] = psum
                ctx_ref[lo:lo + head_dim, :] = pv
            else:
                l_ref[h:h + 1, :] = l_ref[h:h + 1, :] + psum
                ctx_ref[lo:lo + head_dim, :] = ctx_ref[lo:lo + head_dim, :] + pv

    _tile_accum(0, True)
    for j in range(1, n_k):
        @pl.when(j <= qi)
        def _():
            _tile_accum(j, False)

    pieces = []
    for h in range(num_heads):
        lo = h * head_dim
        inv_l = pl.reciprocal(l_ref[h:h + 1, :], approx=False)
        pieces.append(ctx_ref[lo:lo + head_dim, :] * inv_l)
    ctx = jnp.concatenate(pieces, axis=0).astype(jnp.bfloat16)
    out = lax.dot_general(ctx, wo_ref[...], (((0,), (0,)), ((), ())),
                          preferred_element_type=jnp.float32)
    o_ref[0] = (out + bo_ref[...].astype(jnp.float32)).astype(o_ref.dtype)


def kernel(x, wq, wk, wv, wo, bo):
    B, T, d_in = x.shape
    d_out = wq.shape[1]
    num_heads = _NUM_HEADS
    head_dim = d_out // num_heads
    scale = 1.0 / (head_dim ** 0.5)

    wqkvt = jnp.concatenate([wq * scale, wk, wv], axis=1).T.astype(jnp.bfloat16)
    bo2 = bo.reshape(1, d_out)

    t = min(256, _round_up(T, 8))
    tk = t
    T_pad = _round_up(T, t)
    tp = next(c for c in (1024, 512, 256, t) if T_pad % c == 0)
    if T_pad != T:
        x = jnp.pad(x, ((0, 0), (0, T_pad - T), (0, 0)))
    n_p = T_pad // tp
    n_t = T_pad // t

    qkvt = pl.pallas_call(
        _qkvt_kernel,
        out_shape=jax.ShapeDtypeStruct((B, 3 * d_out, T_pad), jnp.bfloat16),
        grid=(B, n_p),
        in_specs=[
            pl.BlockSpec((1, tp, d_in), lambda b, i: (b, i, 0)),
            pl.BlockSpec((3 * d_out, d_in), lambda b, i: (0, 0)),
        ],
        out_specs=pl.BlockSpec((1, 3 * d_out, tp), lambda b, i: (b, 0, i)),
        compiler_params=pltpu.CompilerParams(
            dimension_semantics=("parallel", "parallel")),
    )(x, wqkvt)

    out = pl.pallas_call(
        functools.partial(_attn_kernel, num_heads=num_heads,
                          head_dim=head_dim, tk=tk),
        out_shape=jax.ShapeDtypeStruct((B, T_pad, d_out), x.dtype),
        grid=(B, n_t),
        in_specs=[
            pl.BlockSpec((1, d_out, t), lambda b, qi: (b, 0, qi)),
            pl.BlockSpec((1, d_out, T_pad), lambda b, qi: (b, 1, 0)),
            pl.BlockSpec((1, d_out, T_pad), lambda b, qi: (b, 2, 0)),
            pl.BlockSpec((d_out, d_out), lambda b, qi: (0, 0)),
            pl.BlockSpec((1, d_out), lambda b, qi: (0, 0)),
        ],
        out_specs=pl.BlockSpec((1, t, d_out), lambda b, qi: (b, qi, 0)),
        scratch_shapes=[
            pltpu.VMEM((num_heads, T_pad, t), jnp.float32),
            pltpu.VMEM((T_pad // tk, num_heads, t), jnp.float32),
            pltpu.VMEM((num_heads, t), jnp.float32),
            pltpu.VMEM((d_out, t), jnp.float32),
        ],
        compiler_params=pltpu.CompilerParams(
            dimension_semantics=("parallel", "parallel")),
    )(qkvt, qkvt, qkvt, wo.astype(jnp.bfloat16), bo2)

    if T_pad != T:
        out = out[:, :T, :]
    return out


# speedup vs baseline: 1.0069x; 1.0036x over previous
import functools
import jax
import jax.numpy as jnp
from jax import lax
from jax.experimental import pallas as pl
from jax.experimental.pallas import tpu as pltpu

_NUM_HEADS = 16


def _round_up(x, m):
    return (x + m - 1) // m * m


def _qkvt_kernel(x_ref, wt_ref, qkvt_ref):
    x = x_ref[0].astype(jnp.bfloat16)
    acc = lax.dot_general(wt_ref[...], x, (((1,), (1,)), ((), ())),
                          preferred_element_type=jnp.float32)
    qkvt_ref[0] = acc.astype(qkvt_ref.dtype)


def _attn_kernel(q_ref, k_ref, v_ref, wo_ref, bo_ref, o_ref,
                 st_ref, mt_ref, l_ref, ctx_ref, *, num_heads, head_dim, tk):
    qi = pl.program_id(1)
    tq = q_ref.shape[2]
    T = k_ref.shape[2]
    n_k = T // tk

    qt = q_ref[0]

    neg_diag = jnp.where(
        lax.broadcasted_iota(jnp.int32, (tk, tq), 0)
        > lax.broadcasted_iota(jnp.int32, (tk, tq), 1),
        -jnp.inf, 0.0).astype(jnp.float32)

    mt_ref[...] = jnp.full_like(mt_ref, -jnp.inf)

    def _tile_scores(j, masked):
        for h in range(num_heads):
            lo = h * head_dim
            kt_h = k_ref[0, lo:lo + head_dim, j * tk:(j + 1) * tk]
            st = lax.dot_general(kt_h, qt[lo:lo + head_dim, :],
                                 (((0,), (0,)), ((), ())),
                                 preferred_element_type=jnp.float32)
            if masked:
                st = st + neg_diag
            st_ref[h, j * tk:(j + 1) * tk, :] = st
            mt_ref[j, h:h + 1, :] = jnp.max(st, axis=0, keepdims=True)

    for j in range(n_k):
        if j == 0:
            @pl.when(qi == 0)
            def _():
                _tile_scores(0, True)

            @pl.when(qi > 0)
            def _():
                _tile_scores(0, False)
        else:
            @pl.when(j == qi)
            def _():
                _tile_scores(j, True)

            @pl.when(j < qi)
            def _():
                _tile_scores(j, False)

    m_all = mt_ref[0]
    for j in range(1, n_k):
        m_all = jnp.maximum(m_all, mt_ref[j])

    def _tile_accum(j, first):
        for h in range(num_heads):
            lo = h * head_dim
            p = jnp.exp2(st_ref[h, j * tk:(j + 1) * tk, :]
                         - m_all[h:h + 1, :])
            psum = jnp.sum(p, axis=0, keepdims=True)
            vt_h = v_ref[0, lo:lo + head_dim, j * tk:(j + 1) * tk]
            pv = lax.dot_general(vt_h, p.astype(jnp.bfloat16),
                                 (((1,), (0,)), ((), ())),
                                 preferred_element_type=jnp.float32)
            if first:
                l_ref[h:h + 1, :] = psum
                ctx_ref[lo:lo + head_dim, :] = pv
            else:
                l_ref[h:h + 1, :] = l_ref[h:h + 1, :] + psum
                ctx_ref[lo:lo + head_dim, :] = ctx_ref[lo:lo + head_dim, :] + pv

    _tile_accum(0, True)
    for j in range(1, n_k):
        @pl.when(j <= qi)
        def _():
            _tile_accum(j, False)

    pieces = []
    for h in range(num_heads):
        lo = h * head_dim
        inv_l = pl.reciprocal(l_ref[h:h + 1, :], approx=False)
        pieces.append(ctx_ref[lo:lo + head_dim, :] * inv_l)
    ctx = jnp.concatenate(pieces, axis=0).astype(jnp.bfloat16)
    out = lax.dot_general(ctx, wo_ref[...], (((0,), (0,)), ((), ())),
                          preferred_element_type=jnp.float32)
    o_ref[0] = (out + bo_ref[...].astype(jnp.float32)).astype(o_ref.dtype)


def kernel(x, wq, wk, wv, wo, bo):
    B, T, d_in = x.shape
    d_out = wq.shape[1]
    num_heads = _NUM_HEADS
    head_dim = d_out // num_heads
    scale = 1.0 / (head_dim ** 0.5)

    wqkvt = jnp.concatenate([wq * (scale * 1.4426950408889634), wk, wv],
                            axis=1).T.astype(jnp.bfloat16)
    bo2 = bo.reshape(1, d_out)

    t = min(256, _round_up(T, 8))
    tk = t
    T_pad = _round_up(T, t)
    tp = next(c for c in (1024, 512, 256, t) if T_pad % c == 0)
    if T_pad != T:
        x = jnp.pad(x, ((0, 0), (0, T_pad - T), (0, 0)))
    n_p = T_pad // tp
    n_t = T_pad // t

    qkvt = pl.pallas_call(
        _qkvt_kernel,
        out_shape=jax.ShapeDtypeStruct((B, 3 * d_out, T_pad), jnp.bfloat16),
        grid=(B, n_p),
        in_specs=[
            pl.BlockSpec((1, tp, d_in), lambda b, i: (b, i, 0)),
            pl.BlockSpec((3 * d_out, d_in), lambda b, i: (0, 0)),
        ],
        out_specs=pl.BlockSpec((1, 3 * d_out, tp), lambda b, i: (b, 0, i)),
        compiler_params=pltpu.CompilerParams(
            dimension_semantics=("parallel", "parallel")),
    )(x, wqkvt)

    out = pl.pallas_call(
        functools.partial(_attn_kernel, num_heads=num_heads,
                          head_dim=head_dim, tk=tk),
        out_shape=jax.ShapeDtypeStruct((B, T_pad, d_out), x.dtype),
        grid=(B, n_t),
        in_specs=[
            pl.BlockSpec((1, d_out, t), lambda b, qi: (b, 0, qi)),
            pl.BlockSpec((1, d_out, T_pad), lambda b, qi: (b, 1, 0)),
            pl.BlockSpec((1, d_out, T_pad), lambda b, qi: (b, 2, 0)),
            pl.BlockSpec((d_out, d_out), lambda b, qi: (0, 0)),
            pl.BlockSpec((1, d_out), lambda b, qi: (0, 0)),
        ],
        out_specs=pl.BlockSpec((1, t, d_out), lambda b, qi: (b, qi, 0)),
        scratch_shapes=[
            pltpu.VMEM((num_heads, T_pad, t), jnp.float32),
            pltpu.VMEM((T_pad // tk, num_heads, t), jnp.float32),
            pltpu.VMEM((num_heads, t), jnp.float32),
            pltpu.VMEM((d_out, t), jnp.float32),
        ],
        compiler_params=pltpu.CompilerParams(
            dimension_semantics=("parallel", "parallel")),
    )(qkvt, qkvt, qkvt, wo.astype(jnp.bfloat16), bo2)

    if T_pad != T:
        out = out[:, :T, :]
    return out
